# Initial kernel scaffold; baseline (speedup 1.0000x reference)
import jax
import jax.numpy as jnp
from jax.experimental import pallas as pl


def kernel(x, W1l, W1r, a1, b1, W2l, W2r, a2, b2):
    raise NotImplementedError("write your pallas kernel here")


# speedup vs baseline: 17.0186x; 17.0186x over previous
import functools

import jax
import jax.numpy as jnp
import numpy as np
from jax.experimental import pallas as pl
from jax.experimental.pallas import tpu as pltpu
from jax.experimental.pallas import tpu_sc as plsc

KNN_K = 32
NEG_INF_I32 = np.int32(-(2**31))


def _proj_body(xp_ref, wl_ref, wr_ref, zn_ref, xl_ref, xr_ref):
    xa = xp_ref[...]
    nrm = jnp.sqrt(jnp.sum(xa * xa, axis=1, keepdims=True))
    zn_ref[...] = xa / (nrm + 1e-12)
    xl_ref[...] = jnp.dot(xa, wl_ref[...], preferred_element_type=jnp.float32)
    xr_ref[...] = jnp.dot(xa, wr_ref[...], preferred_element_type=jnp.float32)


def _normalize_and_project(xp, wl, wr):
    npad, d = xp.shape
    f32 = jnp.float32
    return pl.pallas_call(
        _proj_body,
        out_shape=[
            jax.ShapeDtypeStruct((npad, d), f32),
            jax.ShapeDtypeStruct((npad, wl.shape[1]), f32),
            jax.ShapeDtypeStruct((npad, wr.shape[1]), f32),
        ],
    )(xp, wl, wr)


def _topk_body(n_valid, pre_k, znb_ref, znt_ref, idx_ref):
    r = znb_ref.shape[0]
    npad = znt_ref.shape[1]
    vch = npad // 128
    sim = jnp.dot(znb_ref[...], znt_ref[...], preferred_element_type=jnp.float32)
    b = jax.lax.bitcast_convert_type(sim, jnp.int32)
    key = jnp.where(b >= 0, b, b ^ jnp.int32(0x7FFFFFFF)) & jnp.int32(-128)
    s3 = key.reshape(r, vch, 128)
    viota = jax.lax.broadcasted_iota(jnp.int32, (r, vch, 128), 1)
    liota3 = jax.lax.broadcasted_iota(jnp.int32, (r, vch, 128), 2)
    s3 = jnp.where(viota * 128 + liota3 < n_valid, s3 | viota, NEG_INF_I32)
    cands = []
    for p in range(pre_k):
        m = jnp.max(s3, axis=1)
        cands.append(m)
        if p + 1 < pre_k:
            s3 = jnp.where(s3 == m[:, None, :], NEG_INF_I32, s3)
    c = jnp.stack(cands, axis=1)
    lane = jax.lax.broadcasted_iota(jnp.int32, (r, 128), 1)
    acc = jnp.zeros((r, 128), jnp.int32)
    for t in range(KNN_K):
        m8 = jnp.max(c, axis=1)
        m = jnp.max(m8, axis=1, keepdims=True)
        jsel = jnp.min(
            jnp.where(m8 == m, lane, jnp.int32(1 << 20)), axis=1, keepdims=True
        )
        col = (m & jnp.int32(127)) * 128 + jsel
        acc = jnp.where(lane == t, col, acc)
        c = jnp.where(c == m[:, :, None], NEG_INF_I32, c)
    idx_ref[...] = acc[:, :KNN_K]


def _knn_topk(zn, znt, n_valid, row_block=128, pre_k=8):
    npad = zn.shape[0]
    grid = npad // row_block
    return pl.pallas_call(
        functools.partial(_topk_body, n_valid, pre_k),
        grid=(grid,),
        in_specs=[
            pl.BlockSpec((row_block, zn.shape[1]), lambda i: (i, 0)),
            pl.BlockSpec(znt.shape, lambda i: (0, 0)),
        ],
        out_specs=pl.BlockSpec((row_block, KNN_K), lambda i: (i, 0)),
        out_shape=jax.ShapeDtypeStruct((npad, KNN_K), jnp.int32),
    )(zn, znt)


def _sc_gather(table, idx_flat, chunk=128):
    num = idx_flat.shape[0]
    dim = table.shape[1]
    nworkers = 32
    per_w = num // nworkers
    n_chunks = per_w // chunk
    mesh = plsc.VectorSubcoreMesh(core_axis_name="c", subcore_axis_name="s")

    @functools.partial(
        pl.kernel,
        mesh=mesh,
        out_type=jax.ShapeDtypeStruct((num, dim), table.dtype),
        scratch_types=[
            pltpu.VMEM((chunk,), jnp.int32),
            pltpu.VMEM((chunk, dim), table.dtype),
            pltpu.SemaphoreType.DMA,
        ],
    )
    def kern(table_hbm, idx_hbm, out_hbm, idx_v, rows_v, sem):
        wid = jax.lax.axis_index("s") * 2 + jax.lax.axis_index("c")
        base = wid * per_w

        @pl.loop(0, n_chunks)
        def _(c):
            off = base + c * chunk
            pltpu.sync_copy(idx_hbm.at[pl.ds(off, chunk)], idx_v)
            pltpu.async_copy(table_hbm.at[idx_v], rows_v, sem).wait()
            pltpu.sync_copy(rows_v, out_hbm.at[pl.ds(off, chunk)])

    return kern(table, idx_flat)


def _leaky(v):
    return jnp.where(v > 0, v, 0.2 * v)


def _gat1_body(bsz, g_ref, xr_ref, a1s_ref, b1s_ref, b1_ref, w2l_ref, w2r_ref,
               xl2_ref, xr2_ref):
    d = g_ref.shape[1]
    nh = a1s_ref.shape[1]
    g = g_ref[...]
    xr_e = jnp.broadcast_to(
        xr_ref[...][:, None, :], (bsz, KNN_K, d)
    ).reshape(bsz * KNN_K, d)
    e = _leaky(g + xr_e)
    logit = jnp.dot(e, a1s_ref[...], preferred_element_type=jnp.float32)
    ew = jnp.exp(logit).reshape(bsz, KNN_K, nh)
    s = jnp.sum(ew, axis=1, keepdims=True)
    al = (ew / (s + 1e-16)).reshape(bsz * KNN_K, nh)
    alx = jnp.dot(al, b1s_ref[...], preferred_element_type=jnp.float32)
    out = jnp.sum((alx * g).reshape(bsz, KNN_K, d), axis=1)
    h = jnp.maximum(out + b1_ref[...], 0.0)
    xl2_ref[...] = jnp.dot(h, w2l_ref[...], preferred_element_type=jnp.float32)
    xr2_ref[...] = jnp.dot(h, w2r_ref[...], preferred_element_type=jnp.float32)


def _gat_layer1(g1, xr1, a1s, b1s, b1r, w2l, w2r, bsz=256):
    npad, d = xr1.shape
    d2 = w2l.shape[1]
    grid = npad // bsz
    f32 = jnp.float32
    return pl.pallas_call(
        functools.partial(_gat1_body, bsz),
        grid=(grid,),
        in_specs=[
            pl.BlockSpec((bsz * KNN_K, d), lambda i: (i, 0)),
            pl.BlockSpec((bsz, d), lambda i: (i, 0)),
            pl.BlockSpec(a1s.shape, lambda i: (0, 0)),
            pl.BlockSpec(b1s.shape, lambda i: (0, 0)),
            pl.BlockSpec(b1r.shape, lambda i: (0, 0)),
            pl.BlockSpec(w2l.shape, lambda i: (0, 0)),
            pl.BlockSpec(w2r.shape, lambda i: (0, 0)),
        ],
        out_specs=[
            pl.BlockSpec((bsz, d2), lambda i: (i, 0)),
            pl.BlockSpec((bsz, d2), lambda i: (i, 0)),
        ],
        out_shape=[
            jax.ShapeDtypeStruct((npad, d2), f32),
            jax.ShapeDtypeStruct((npad, d2), f32),
        ],
    )(g1, xr1, a1s, b1s, b1r, w2l, w2r)


def _gat2_body(bsz, heads, g_ref, xr_ref, a2s_ref, b2s_ref, b2_ref, z_ref):
    d2 = g_ref.shape[1]
    nh = a2s_ref.shape[1]
    oc = d2 // heads
    g = g_ref[...]
    xr_e = jnp.broadcast_to(
        xr_ref[...][:, None, :], (bsz, KNN_K, d2)
    ).reshape(bsz * KNN_K, d2)
    e = _leaky(g + xr_e)
    logit = jnp.dot(e, a2s_ref[...], preferred_element_type=jnp.float32)
    ew = jnp.exp(logit).reshape(bsz, KNN_K, nh)
    s = jnp.sum(ew, axis=1, keepdims=True)
    al = (ew / (s + 1e-16)).reshape(bsz * KNN_K, nh)
    alx = jnp.dot(al, b2s_ref[...], preferred_element_type=jnp.float32)
    w = jnp.sum((alx * g).reshape(bsz, KNN_K, d2), axis=1)
    acc = w[:, 0:oc]
    for hh in range(1, heads):
        acc = acc + w[:, hh * oc:(hh + 1) * oc]
    z_ref[...] = acc * (1.0 / heads) + b2_ref[...]


def _gat_layer2(g2, xr2, a2s, b2s, b2r, heads, bsz=64):
    npad, d2 = xr2.shape
    oc = d2 // heads
    grid = npad // bsz
    return pl.pallas_call(
        functools.partial(_gat2_body, bsz, heads),
        grid=(grid,),
        in_specs=[
            pl.BlockSpec((bsz * KNN_K, d2), lambda i: (i, 0)),
            pl.BlockSpec((bsz, d2), lambda i: (i, 0)),
            pl.BlockSpec(a2s.shape, lambda i: (0, 0)),
            pl.BlockSpec(b2s.shape, lambda i: (0, 0)),
            pl.BlockSpec(b2r.shape, lambda i: (0, 0)),
        ],
        out_specs=pl.BlockSpec((bsz, oc), lambda i: (i, 0)),
        out_shape=jax.ShapeDtypeStruct((npad, oc), jnp.float32),
    )(g2, xr2, a2s, b2s, b2r)


def _spread_att(a, nh_pad=8):
    heads, oc = a.shape
    eye = jnp.eye(heads, nh_pad, dtype=a.dtype)
    return (a[:, :, None] * eye[:, None, :]).reshape(heads * oc, nh_pad)


def _head_broadcast(heads, oc, nh_pad=8):
    return jnp.repeat(jnp.eye(nh_pad, heads, dtype=jnp.float32), oc, axis=1)


def kernel(x, W1l, W1r, a1, b1, W2l, W2r, a2, b2):
    n, d = x.shape
    heads, oc1 = a1.shape
    npad = ((n + 1023) // 1024) * 1024
    xp = jnp.pad(x, ((0, npad - n), (0, 0)))

    zn, xl1, xr1 = _normalize_and_project(xp, W1l, W1r)
    idx = _knn_topk(zn, zn.T, n)
    idx_flat = idx.reshape(-1)

    g1 = _sc_gather(xl1, idx_flat)
    a1s = _spread_att(a1)
    b1s = _head_broadcast(heads, oc1)
    xl2, xr2 = _gat_layer1(g1, xr1, a1s, b1s, b1.reshape(1, -1), W2l, W2r)

    d2 = xl2.shape[1]
    idx2_flat = (idx_flat[:, None] * 2
                 + jnp.arange(2, dtype=jnp.int32)[None, :]).reshape(-1)
    g2 = _sc_gather(xl2.reshape(npad * 2, d2 // 2), idx2_flat)
    g2 = g2.reshape(idx_flat.shape[0], d2)
    a2s = _spread_att(a2)
    b2s = _head_broadcast(heads, a2.shape[1])
    z = _gat_layer2(g2, xr2, a2s, b2s, b2.reshape(1, -1), heads)
    return (x, z[:n])


# speedup vs baseline: 23.1109x; 1.3580x over previous
import functools

import jax
import jax.numpy as jnp
import numpy as np
from jax.experimental import pallas as pl
from jax.experimental.pallas import tpu as pltpu
from jax.experimental.pallas import tpu_sc as plsc

KNN_K = 32
NEG_INF_I32 = np.int32(-(2**31))


def _proj_body(xp_ref, wl_ref, wr_ref, zn_ref, xl_ref, xr_ref):
    xa = xp_ref[...]
    nrm = jnp.sqrt(jnp.sum(xa * xa, axis=1, keepdims=True))
    zn_ref[...] = xa / (nrm + 1e-12)
    xl_ref[...] = jnp.dot(xa, wl_ref[...], preferred_element_type=jnp.float32)
    xr_ref[...] = jnp.dot(xa, wr_ref[...], preferred_element_type=jnp.float32)


def _normalize_and_project(xp, wl, wr):
    npad, d = xp.shape
    f32 = jnp.float32
    return pl.pallas_call(
        _proj_body,
        out_shape=[
            jax.ShapeDtypeStruct((npad, d), f32),
            jax.ShapeDtypeStruct((npad, wl.shape[1]), f32),
            jax.ShapeDtypeStruct((npad, wr.shape[1]), f32),
        ],
    )(xp, wl, wr)


def _topk_body(n_valid, pre_k, znb_ref, znt_ref, idx_ref):
    r = znb_ref.shape[0]
    npad = znt_ref.shape[1]
    vch = npad // 128
    sim = jnp.dot(znb_ref[...], znt_ref[...], preferred_element_type=jnp.float32)
    b = jax.lax.bitcast_convert_type(sim, jnp.int32)
    key = jnp.where(b >= 0, b, b ^ jnp.int32(0x7FFFFFFF)) & jnp.int32(-128)
    s3 = key.reshape(r, vch, 128)
    viota = jax.lax.broadcasted_iota(jnp.int32, (r, vch, 128), 1)
    liota3 = jax.lax.broadcasted_iota(jnp.int32, (r, vch, 128), 2)
    s3 = jnp.where(viota * 128 + liota3 < n_valid, s3 | viota, NEG_INF_I32)
    cands = []
    for p in range(pre_k):
        m = jnp.max(s3, axis=1)
        cands.append(m)
        if p + 1 < pre_k:
            s3 = jnp.where(s3 == m[:, None, :], NEG_INF_I32, s3)
    c = jnp.stack(cands, axis=1)
    lane = jax.lax.broadcasted_iota(jnp.int32, (r, 128), 1)
    acc = jnp.zeros((r, 128), jnp.int32)
    for t in range(KNN_K):
        m8 = jnp.max(c, axis=1)
        m = jnp.max(m8, axis=1, keepdims=True)
        jsel = jnp.min(
            jnp.where(m8 == m, lane, jnp.int32(1 << 20)), axis=1, keepdims=True
        )
        col = (m & jnp.int32(127)) * 128 + jsel
        acc = jnp.where(lane == t, col, acc)
        c = jnp.where(c == m[:, :, None], NEG_INF_I32, c)
    idx_ref[...] = acc[:, :KNN_K]


def _knn_topk(zn, znt, n_valid, row_block=128, pre_k=8):
    npad = zn.shape[0]
    grid = npad // row_block
    return pl.pallas_call(
        functools.partial(_topk_body, n_valid, pre_k),
        grid=(grid,),
        in_specs=[
            pl.BlockSpec((row_block, zn.shape[1]), lambda i: (i, 0)),
            pl.BlockSpec(znt.shape, lambda i: (0, 0)),
        ],
        out_specs=pl.BlockSpec((row_block, KNN_K), lambda i: (i, 0)),
        out_shape=jax.ShapeDtypeStruct((npad, KNN_K), jnp.int32),
    )(zn, znt)


def _sc_gather(table, idx_flat, chunk=128):
    num = idx_flat.shape[0]
    dim = table.shape[1]
    nworkers = 32
    per_w = num // nworkers
    n_chunks = per_w // chunk
    mesh = plsc.VectorSubcoreMesh(core_axis_name="c", subcore_axis_name="s")

    @functools.partial(
        pl.kernel,
        mesh=mesh,
        out_type=jax.ShapeDtypeStruct((num, dim), table.dtype),
        scratch_types=[
            pltpu.VMEM((chunk,), jnp.int32),
            pltpu.VMEM((chunk, dim), table.dtype),
            pltpu.SemaphoreType.DMA,
        ],
    )
    def kern(table_hbm, idx_hbm, out_hbm, idx_v, rows_v, sem):
        wid = jax.lax.axis_index("s") * 2 + jax.lax.axis_index("c")
        base = wid * per_w

        @pl.loop(0, n_chunks)
        def _(c):
            off = base + c * chunk
            pltpu.sync_copy(idx_hbm.at[pl.ds(off, chunk)], idx_v)
            pltpu.async_copy(table_hbm.at[idx_v], rows_v, sem).wait()
            pltpu.sync_copy(rows_v, out_hbm.at[pl.ds(off, chunk)])

    return kern(table, idx_flat)


def _leaky(v):
    return jnp.where(v > 0, v, 0.2 * v)


def _gat1_body(bsz, g_ref, xr_ref, a1s_ref, b1s_ref, b1_ref, w2r_ref,
               h_ref, xr2_ref):
    d = g_ref.shape[1]
    nh = a1s_ref.shape[1]
    g = g_ref[...]
    xr_e = jnp.broadcast_to(
        xr_ref[...][:, None, :], (bsz, KNN_K, d)
    ).reshape(bsz * KNN_K, d)
    e = _leaky(g + xr_e)
    logit = jnp.dot(e, a1s_ref[...], preferred_element_type=jnp.float32)
    ew = jnp.exp(logit).reshape(bsz, KNN_K, nh)
    s = jnp.sum(ew, axis=1, keepdims=True)
    al = (ew / (s + 1e-16)).reshape(bsz * KNN_K, nh)
    alx = jnp.dot(al, b1s_ref[...], preferred_element_type=jnp.float32)
    out = jnp.sum((alx * g).reshape(bsz, KNN_K, d), axis=1)
    h = jnp.maximum(out + b1_ref[...], 0.0)
    h_ref[...] = h
    xr2_ref[...] = jnp.dot(h, w2r_ref[...], preferred_element_type=jnp.float32)


def _gat_layer1(g1, xr1, a1s, b1s, b1r, w2r, bsz=256):
    npad, d = xr1.shape
    d2 = w2r.shape[1]
    grid = npad // bsz
    f32 = jnp.float32
    return pl.pallas_call(
        functools.partial(_gat1_body, bsz),
        grid=(grid,),
        in_specs=[
            pl.BlockSpec((bsz * KNN_K, d), lambda i: (i, 0)),
            pl.BlockSpec((bsz, d), lambda i: (i, 0)),
            pl.BlockSpec(a1s.shape, lambda i: (0, 0)),
            pl.BlockSpec(b1s.shape, lambda i: (0, 0)),
            pl.BlockSpec(b1r.shape, lambda i: (0, 0)),
            pl.BlockSpec(w2r.shape, lambda i: (0, 0)),
        ],
        out_specs=[
            pl.BlockSpec((bsz, d), lambda i: (i, 0)),
            pl.BlockSpec((bsz, d2), lambda i: (i, 0)),
        ],
        out_shape=[
            jax.ShapeDtypeStruct((npad, d), f32),
            jax.ShapeDtypeStruct((npad, d2), f32),
        ],
    )(g1, xr1, a1s, b1s, b1r, w2r)


def _gat2_body(bsz, heads, gh_ref, w2l_ref, xr_ref, a2s_ref, b2s_ref, b2_ref,
               z_ref):
    d2 = xr_ref.shape[1]
    nh = a2s_ref.shape[1]
    oc = d2 // heads
    g = jnp.dot(gh_ref[...], w2l_ref[...], preferred_element_type=jnp.float32)
    xr_e = jnp.broadcast_to(
        xr_ref[...][:, None, :], (bsz, KNN_K, d2)
    ).reshape(bsz * KNN_K, d2)
    e = _leaky(g + xr_e)
    logit = jnp.dot(e, a2s_ref[...], preferred_element_type=jnp.float32)
    ew = jnp.exp(logit).reshape(bsz, KNN_K, nh)
    s = jnp.sum(ew, axis=1, keepdims=True)
    al = (ew / (s + 1e-16)).reshape(bsz * KNN_K, nh)
    alx = jnp.dot(al, b2s_ref[...], preferred_element_type=jnp.float32)
    w = jnp.sum((alx * g).reshape(bsz, KNN_K, d2), axis=1)
    acc = w[:, 0:oc]
    for hh in range(1, heads):
        acc = acc + w[:, hh * oc:(hh + 1) * oc]
    z_ref[...] = acc * (1.0 / heads) + b2_ref[...]


def _gat_layer2(g2h, w2l, xr2, a2s, b2s, b2r, heads, bsz=128):
    npad, d2 = xr2.shape
    d = g2h.shape[1]
    oc = d2 // heads
    grid = npad // bsz
    return pl.pallas_call(
        functools.partial(_gat2_body, bsz, heads),
        grid=(grid,),
        in_specs=[
            pl.BlockSpec((bsz * KNN_K, d), lambda i: (i, 0)),
            pl.BlockSpec(w2l.shape, lambda i: (0, 0)),
            pl.BlockSpec((bsz, d2), lambda i: (i, 0)),
            pl.BlockSpec(a2s.shape, lambda i: (0, 0)),
            pl.BlockSpec(b2s.shape, lambda i: (0, 0)),
            pl.BlockSpec(b2r.shape, lambda i: (0, 0)),
        ],
        out_specs=pl.BlockSpec((bsz, oc), lambda i: (i, 0)),
        out_shape=jax.ShapeDtypeStruct((npad, oc), jnp.float32),
    )(g2h, w2l, xr2, a2s, b2s, b2r)


def _spread_att(a, nh_pad=8):
    heads, oc = a.shape
    eye = jnp.eye(heads, nh_pad, dtype=a.dtype)
    return (a[:, :, None] * eye[:, None, :]).reshape(heads * oc, nh_pad)


def _head_broadcast(heads, oc, nh_pad=8):
    return jnp.repeat(jnp.eye(nh_pad, heads, dtype=jnp.float32), oc, axis=1)


def kernel(x, W1l, W1r, a1, b1, W2l, W2r, a2, b2):
    n, d = x.shape
    heads, oc1 = a1.shape
    npad = ((n + 1023) // 1024) * 1024
    xp = jnp.pad(x, ((0, npad - n), (0, 0)))

    zn, xl1, xr1 = _normalize_and_project(xp, W1l, W1r)
    idx = _knn_topk(zn, zn.T, n)
    idx_flat = idx.reshape(-1)

    g1 = _sc_gather(xl1, idx_flat)
    a1s = _spread_att(a1)
    b1s = _head_broadcast(heads, oc1)
    h, xr2 = _gat_layer1(g1, xr1, a1s, b1s, b1.reshape(1, -1), W2r)

    g2h = _sc_gather(h, idx_flat)
    a2s = _spread_att(a2)
    b2s = _head_broadcast(heads, a2.shape[1])
    z = _gat_layer2(g2h, W2l, xr2, a2s, b2s, b2.reshape(1, -1), heads)
    return (x, z[:n])


# speedup vs baseline: 25.0826x; 1.0853x over previous
import functools

import jax
import jax.numpy as jnp
import numpy as np
from jax.experimental import pallas as pl
from jax.experimental.pallas import tpu as pltpu
from jax.experimental.pallas import tpu_sc as plsc

KNN_K = 32
NEG_INF_I32 = np.int32(-(2**31))


def _proj_body(xp_ref, wl_ref, wr_ref, zn_ref, xl_ref, xr_ref):
    xa = xp_ref[...]
    nrm = jnp.sqrt(jnp.sum(xa * xa, axis=1, keepdims=True))
    zn_ref[...] = xa / (nrm + 1e-12)
    xl_ref[...] = jnp.dot(xa, wl_ref[...], preferred_element_type=jnp.float32)
    xr_ref[...] = jnp.dot(xa, wr_ref[...], preferred_element_type=jnp.float32)


def _normalize_and_project(xp, wl, wr):
    npad, d = xp.shape
    f32 = jnp.float32
    return pl.pallas_call(
        _proj_body,
        out_shape=[
            jax.ShapeDtypeStruct((npad, d), f32),
            jax.ShapeDtypeStruct((npad, wl.shape[1]), f32),
            jax.ShapeDtypeStruct((npad, wr.shape[1]), f32),
        ],
    )(xp, wl, wr)


def _topk_body(n_valid, pre_k, znb_ref, znt_ref, idx_ref, key_ref):
    r = znb_ref.shape[0]
    npad = znt_ref.shape[1]
    sim = jnp.dot(znb_ref[...], znt_ref[...], preferred_element_type=jnp.float32)
    b = jax.lax.bitcast_convert_type(sim, jnp.int32)
    key_ref[...] = jnp.where(b >= 0, b, b ^ jnp.int32(0x7FFFFFFF)) & jnp.int32(-128)
    lane = jax.lax.broadcasted_iota(jnp.int32, (r, 128), 1)
    full_ch = n_valid // 128
    part = n_valid - full_ch * 128

    def _insert(xv, regs):
        out = []
        for i in range(pre_k):
            out.append(jnp.maximum(regs[i], xv))
            if i + 1 < pre_k:
                xv = jnp.minimum(regs[i], xv)
        return tuple(out)

    def _chunk_body(v, regs):
        xv = key_ref[:, pl.ds(v * 128, 128)] | v
        return _insert(xv, regs)

    regs = tuple(
        jnp.full((r, 128), NEG_INF_I32, jnp.int32) for _ in range(pre_k)
    )
    regs = jax.lax.fori_loop(0, full_ch, _chunk_body, regs)
    if part:
        xv = key_ref[:, pl.ds(full_ch * 128, 128)] | jnp.int32(full_ch)
        xv = jnp.where(lane < part, xv, NEG_INF_I32)
        regs = _insert(xv, regs)
    c = jnp.stack(regs, axis=1)
    neg_tail = jnp.full((r, 1, 128), NEG_INF_I32, jnp.int32)
    acc = jnp.zeros((r, 128), jnp.int32)
    for t in range(KNN_K):
        head = c[:, 0, :]
        m = jnp.max(head, axis=1, keepdims=True)
        jsel = jnp.min(
            jnp.where(head == m, lane, jnp.int32(1 << 20)), axis=1, keepdims=True
        )
        col = (m & jnp.int32(127)) * 128 + jsel
        acc = jnp.where(lane == t, col, acc)
        if t + 1 < KNN_K:
            shifted = jnp.concatenate([c[:, 1:, :], neg_tail], axis=1)
            c = jnp.where((lane == jsel)[:, None, :], shifted, c)
    idx_ref[...] = acc[:, :KNN_K]


def _knn_topk(zn, znt, n_valid, row_block=128, pre_k=8):
    npad = zn.shape[0]
    grid = npad // row_block
    return pl.pallas_call(
        functools.partial(_topk_body, n_valid, pre_k),
        grid=(grid,),
        in_specs=[
            pl.BlockSpec((row_block, zn.shape[1]), lambda i: (i, 0)),
            pl.BlockSpec(znt.shape, lambda i: (0, 0)),
        ],
        out_specs=pl.BlockSpec((row_block, KNN_K), lambda i: (i, 0)),
        out_shape=jax.ShapeDtypeStruct((npad, KNN_K), jnp.int32),
        scratch_shapes=[pltpu.VMEM((row_block, npad), jnp.int32)],
    )(zn, znt)


def _sc_gather(table, idx_flat, chunk=128):
    num = idx_flat.shape[0]
    dim = table.shape[1]
    nworkers = 32
    per_w = num // nworkers
    n_chunks = per_w // chunk
    mesh = plsc.VectorSubcoreMesh(core_axis_name="c", subcore_axis_name="s")

    @functools.partial(
        pl.kernel,
        mesh=mesh,
        out_type=jax.ShapeDtypeStruct((num, dim), table.dtype),
        scratch_types=[
            pltpu.VMEM((chunk,), jnp.int32),
            pltpu.VMEM((chunk, dim), table.dtype),
            pltpu.SemaphoreType.DMA,
        ],
    )
    def kern(table_hbm, idx_hbm, out_hbm, idx_v, rows_v, sem):
        wid = jax.lax.axis_index("s") * 2 + jax.lax.axis_index("c")
        base = wid * per_w

        @pl.loop(0, n_chunks)
        def _(c):
            off = base + c * chunk
            pltpu.sync_copy(idx_hbm.at[pl.ds(off, chunk)], idx_v)
            pltpu.async_copy(table_hbm.at[idx_v], rows_v, sem).wait()
            pltpu.sync_copy(rows_v, out_hbm.at[pl.ds(off, chunk)])

    return kern(table, idx_flat)


def _leaky(v):
    return jnp.where(v > 0, v, 0.2 * v)


def _gat1_body(bsz, g_ref, xr_ref, a1s_ref, b1s_ref, b1_ref, w2r_ref,
               h_ref, xr2_ref):
    d = g_ref.shape[1]
    nh = a1s_ref.shape[1]
    g = g_ref[...]
    xr_e = jnp.broadcast_to(
        xr_ref[...][:, None, :], (bsz, KNN_K, d)
    ).reshape(bsz * KNN_K, d)
    e = _leaky(g + xr_e)
    logit = jnp.dot(e, a1s_ref[...], preferred_element_type=jnp.float32)
    ew = jnp.exp(logit).reshape(bsz, KNN_K, nh)
    s = jnp.sum(ew, axis=1, keepdims=True)
    al = (ew / (s + 1e-16)).reshape(bsz * KNN_K, nh)
    alx = jnp.dot(al, b1s_ref[...], preferred_element_type=jnp.float32)
    out = jnp.sum((alx * g).reshape(bsz, KNN_K, d), axis=1)
    h = jnp.maximum(out + b1_ref[...], 0.0)
    h_ref[...] = h
    xr2_ref[...] = jnp.dot(h, w2r_ref[...], preferred_element_type=jnp.float32)


def _gat_layer1(g1, xr1, a1s, b1s, b1r, w2r, bsz=256):
    npad, d = xr1.shape
    d2 = w2r.shape[1]
    grid = npad // bsz
    f32 = jnp.float32
    return pl.pallas_call(
        functools.partial(_gat1_body, bsz),
        grid=(grid,),
        in_specs=[
            pl.BlockSpec((bsz * KNN_K, d), lambda i: (i, 0)),
            pl.BlockSpec((bsz, d), lambda i: (i, 0)),
            pl.BlockSpec(a1s.shape, lambda i: (0, 0)),
            pl.BlockSpec(b1s.shape, lambda i: (0, 0)),
            pl.BlockSpec(b1r.shape, lambda i: (0, 0)),
            pl.BlockSpec(w2r.shape, lambda i: (0, 0)),
        ],
        out_specs=[
            pl.BlockSpec((bsz, d), lambda i: (i, 0)),
            pl.BlockSpec((bsz, d2), lambda i: (i, 0)),
        ],
        out_shape=[
            jax.ShapeDtypeStruct((npad, d), f32),
            jax.ShapeDtypeStruct((npad, d2), f32),
        ],
    )(g1, xr1, a1s, b1s, b1r, w2r)


def _gat2_body(bsz, heads, gh_ref, w2l_ref, xr_ref, a2s_ref, b2s_ref, b2_ref,
               z_ref):
    d2 = xr_ref.shape[1]
    nh = a2s_ref.shape[1]
    oc = d2 // heads
    g = jnp.dot(gh_ref[...], w2l_ref[...], preferred_element_type=jnp.float32)
    xr_e = jnp.broadcast_to(
        xr_ref[...][:, None, :], (bsz, KNN_K, d2)
    ).reshape(bsz * KNN_K, d2)
    e = _leaky(g + xr_e)
    logit = jnp.dot(e, a2s_ref[...], preferred_element_type=jnp.float32)
    ew = jnp.exp(logit).reshape(bsz, KNN_K, nh)
    s = jnp.sum(ew, axis=1, keepdims=True)
    al = (ew / (s + 1e-16)).reshape(bsz * KNN_K, nh)
    alx = jnp.dot(al, b2s_ref[...], preferred_element_type=jnp.float32)
    w = jnp.sum((alx * g).reshape(bsz, KNN_K, d2), axis=1)
    acc = w[:, 0:oc]
    for hh in range(1, heads):
        acc = acc + w[:, hh * oc:(hh + 1) * oc]
    z_ref[...] = acc * (1.0 / heads) + b2_ref[...]


def _gat_layer2(g2h, w2l, xr2, a2s, b2s, b2r, heads, bsz=128):
    npad, d2 = xr2.shape
    d = g2h.shape[1]
    oc = d2 // heads
    grid = npad // bsz
    return pl.pallas_call(
        functools.partial(_gat2_body, bsz, heads),
        grid=(grid,),
        in_specs=[
            pl.BlockSpec((bsz * KNN_K, d), lambda i: (i, 0)),
            pl.BlockSpec(w2l.shape, lambda i: (0, 0)),
            pl.BlockSpec((bsz, d2), lambda i: (i, 0)),
            pl.BlockSpec(a2s.shape, lambda i: (0, 0)),
            pl.BlockSpec(b2s.shape, lambda i: (0, 0)),
            pl.BlockSpec(b2r.shape, lambda i: (0, 0)),
        ],
        out_specs=pl.BlockSpec((bsz, oc), lambda i: (i, 0)),
        out_shape=jax.ShapeDtypeStruct((npad, oc), jnp.float32),
    )(g2h, w2l, xr2, a2s, b2s, b2r)


def _spread_att(a, nh_pad=8):
    heads, oc = a.shape
    eye = jnp.eye(heads, nh_pad, dtype=a.dtype)
    return (a[:, :, None] * eye[:, None, :]).reshape(heads * oc, nh_pad)


def _head_broadcast(heads, oc, nh_pad=8):
    return jnp.repeat(jnp.eye(nh_pad, heads, dtype=jnp.float32), oc, axis=1)


def kernel(x, W1l, W1r, a1, b1, W2l, W2r, a2, b2):
    n, d = x.shape
    heads, oc1 = a1.shape
    npad = ((n + 1023) // 1024) * 1024
    xp = jnp.pad(x, ((0, npad - n), (0, 0)))

    zn, xl1, xr1 = _normalize_and_project(xp, W1l, W1r)
    idx = _knn_topk(zn, zn.T, n)
    idx_flat = idx.reshape(-1)

    g1 = _sc_gather(xl1, idx_flat)
    a1s = _spread_att(a1)
    b1s = _head_broadcast(heads, oc1)
    h, xr2 = _gat_layer1(g1, xr1, a1s, b1s, b1.reshape(1, -1), W2r)

    g2h = _sc_gather(h, idx_flat)
    a2s = _spread_att(a2)
    b2s = _head_broadcast(heads, a2.shape[1])
    z = _gat_layer2(g2h, W2l, xr2, a2s, b2s, b2.reshape(1, -1), heads)
    return (x, z[:n])


# speedup vs baseline: 26.3268x; 1.0496x over previous
import functools

import jax
import jax.numpy as jnp
import numpy as np
from jax.experimental import pallas as pl
from jax.experimental.pallas import tpu as pltpu
from jax.experimental.pallas import tpu_sc as plsc

KNN_K = 32
NEG_INF_I32 = np.int32(-(2**31))


def _proj_body(xp_ref, wl_ref, wr_ref, zn_ref, xl_ref, xr_ref):
    xa = xp_ref[...]
    nrm = jnp.sqrt(jnp.sum(xa * xa, axis=1, keepdims=True))
    zn_ref[...] = xa / (nrm + 1e-12)
    xl_ref[...] = jnp.dot(xa, wl_ref[...], preferred_element_type=jnp.float32)
    xr_ref[...] = jnp.dot(xa, wr_ref[...], preferred_element_type=jnp.float32)


def _normalize_and_project(xp, wl, wr):
    npad, d = xp.shape
    f32 = jnp.float32
    return pl.pallas_call(
        _proj_body,
        out_shape=[
            jax.ShapeDtypeStruct((npad, d), f32),
            jax.ShapeDtypeStruct((npad, wl.shape[1]), f32),
            jax.ShapeDtypeStruct((npad, wr.shape[1]), f32),
        ],
    )(xp, wl, wr)


def _topk_body(n_valid, pre_k, znb_ref, znt_ref, idx_ref, key_ref):
    r = znb_ref.shape[0]
    npad = znt_ref.shape[1]
    sim = jnp.dot(znb_ref[...], znt_ref[...], preferred_element_type=jnp.float32)
    b = jax.lax.bitcast_convert_type(sim, jnp.int32)
    key_ref[...] = jnp.where(b >= 0, b, b ^ jnp.int32(0x7FFFFFFF)) & jnp.int32(-128)
    lane = jax.lax.broadcasted_iota(jnp.int32, (r, 128), 1)
    full_ch = n_valid // 128
    part = n_valid - full_ch * 128

    def _insert(xv, regs):
        out = []
        for i in range(pre_k):
            out.append(jnp.maximum(regs[i], xv))
            if i + 1 < pre_k:
                xv = jnp.minimum(regs[i], xv)
        return tuple(out)

    def _chunk_body(v, regs):
        xv = key_ref[:, pl.ds(v * 128, 128)] | v
        return _insert(xv, regs)

    regs = tuple(
        jnp.full((r, 128), NEG_INF_I32, jnp.int32) for _ in range(pre_k)
    )
    regs = jax.lax.fori_loop(0, full_ch, _chunk_body, regs)
    if part:
        xv = key_ref[:, pl.ds(full_ch * 128, 128)] | jnp.int32(full_ch)
        xv = jnp.where(lane < part, xv, NEG_INF_I32)
        regs = _insert(xv, regs)
    c = jnp.stack(regs, axis=1)
    neg_tail = jnp.full((r, 1, 128), NEG_INF_I32, jnp.int32)
    acc = jnp.zeros((r, 128), jnp.int32)
    for t in range(KNN_K):
        head = c[:, 0, :]
        m = jnp.max(head, axis=1, keepdims=True)
        jsel = jnp.min(
            jnp.where(head == m, lane, jnp.int32(1 << 20)), axis=1, keepdims=True
        )
        col = (m & jnp.int32(127)) * 128 + jsel
        acc = jnp.where(lane == t, col, acc)
        if t + 1 < KNN_K:
            shifted = jnp.concatenate([c[:, 1:, :], neg_tail], axis=1)
            c = jnp.where((lane == jsel)[:, None, :], shifted, c)
    idx_ref[...] = acc[:, :KNN_K]


def _knn_topk(zn, znt, n_valid, row_block=128, pre_k=8):
    npad = zn.shape[0]
    grid = npad // row_block
    return pl.pallas_call(
        functools.partial(_topk_body, n_valid, pre_k),
        grid=(grid,),
        in_specs=[
            pl.BlockSpec((row_block, zn.shape[1]), lambda i: (i, 0)),
            pl.BlockSpec(znt.shape, lambda i: (0, 0)),
        ],
        out_specs=pl.BlockSpec((row_block, KNN_K), lambda i: (i, 0)),
        out_shape=jax.ShapeDtypeStruct((npad, KNN_K), jnp.int32),
        scratch_shapes=[pltpu.VMEM((row_block, npad), jnp.int32)],
    )(zn, znt)


def _sc_gather(table, idx_flat, chunk=128, nslot=4):
    num = idx_flat.shape[0]
    dim = table.shape[1]
    nworkers = 32
    per_w = num // nworkers
    sup = chunk * nslot
    n_sup = per_w // sup
    mesh = plsc.VectorSubcoreMesh(core_axis_name="c", subcore_axis_name="s")

    @functools.partial(
        pl.kernel,
        mesh=mesh,
        out_type=jax.ShapeDtypeStruct((num, dim), table.dtype),
        scratch_types=[
            pltpu.VMEM((sup,), jnp.int32),
            pltpu.VMEM((sup, dim), table.dtype),
            pltpu.SemaphoreType.DMA,
        ],
    )
    def kern(table_hbm, idx_hbm, out_hbm, idx_v, rows_v, sem):
        wid = jax.lax.axis_index("s") * 2 + jax.lax.axis_index("c")
        base = wid * per_w

        @pl.loop(0, n_sup)
        def _(c):
            off = base + c * sup
            pltpu.sync_copy(idx_hbm.at[pl.ds(off, sup)], idx_v)
            cps = [
                pltpu.async_copy(
                    table_hbm.at[idx_v.at[pl.ds(k * chunk, chunk)]],
                    rows_v.at[pl.ds(k * chunk, chunk)],
                    sem,
                )
                for k in range(nslot)
            ]
            for cp in cps:
                cp.wait()
            pltpu.sync_copy(rows_v, out_hbm.at[pl.ds(off, sup)])

    return kern(table, idx_flat)


def _leaky(v):
    return jnp.where(v > 0, v, 0.2 * v)


def _gat1_body(bsz, g_ref, xr_ref, a1s_ref, b1s_ref, b1_ref, w2r_ref,
               h_ref, xr2_ref):
    d = g_ref.shape[1]
    nh = a1s_ref.shape[1]
    g = g_ref[...]
    xr_e = jnp.broadcast_to(
        xr_ref[...][:, None, :], (bsz, KNN_K, d)
    ).reshape(bsz * KNN_K, d)
    e = _leaky(g + xr_e)
    logit = jnp.dot(e, a1s_ref[...], preferred_element_type=jnp.float32)
    ew = jnp.exp(logit).reshape(bsz, KNN_K, nh)
    s = jnp.sum(ew, axis=1, keepdims=True)
    al = (ew / (s + 1e-16)).reshape(bsz * KNN_K, nh)
    alx = jnp.dot(al, b1s_ref[...], preferred_element_type=jnp.float32)
    out = jnp.sum((alx * g).reshape(bsz, KNN_K, d), axis=1)
    h = jnp.maximum(out + b1_ref[...], 0.0)
    h_ref[...] = h
    xr2_ref[...] = jnp.dot(h, w2r_ref[...], preferred_element_type=jnp.float32)


def _gat_layer1(g1, xr1, a1s, b1s, b1r, w2r, bsz=256):
    npad, d = xr1.shape
    d2 = w2r.shape[1]
    grid = npad // bsz
    f32 = jnp.float32
    return pl.pallas_call(
        functools.partial(_gat1_body, bsz),
        grid=(grid,),
        in_specs=[
            pl.BlockSpec((bsz * KNN_K, d), lambda i: (i, 0)),
            pl.BlockSpec((bsz, d), lambda i: (i, 0)),
            pl.BlockSpec(a1s.shape, lambda i: (0, 0)),
            pl.BlockSpec(b1s.shape, lambda i: (0, 0)),
            pl.BlockSpec(b1r.shape, lambda i: (0, 0)),
            pl.BlockSpec(w2r.shape, lambda i: (0, 0)),
        ],
        out_specs=[
            pl.BlockSpec((bsz, d), lambda i: (i, 0)),
            pl.BlockSpec((bsz, d2), lambda i: (i, 0)),
        ],
        out_shape=[
            jax.ShapeDtypeStruct((npad, d), f32),
            jax.ShapeDtypeStruct((npad, d2), f32),
        ],
    )(g1, xr1, a1s, b1s, b1r, w2r)


def _gat2_body(bsz, heads, gh_ref, w2l_ref, xr_ref, a2s_ref, b2s_ref, b2_ref,
               z_ref):
    d2 = xr_ref.shape[1]
    nh = a2s_ref.shape[1]
    oc = d2 // heads
    g = jnp.dot(gh_ref[...], w2l_ref[...], preferred_element_type=jnp.float32)
    xr_e = jnp.broadcast_to(
        xr_ref[...][:, None, :], (bsz, KNN_K, d2)
    ).reshape(bsz * KNN_K, d2)
    e = _leaky(g + xr_e)
    logit = jnp.dot(e, a2s_ref[...], preferred_element_type=jnp.float32)
    ew = jnp.exp(logit).reshape(bsz, KNN_K, nh)
    s = jnp.sum(ew, axis=1, keepdims=True)
    al = (ew / (s + 1e-16)).reshape(bsz * KNN_K, nh)
    alx = jnp.dot(al, b2s_ref[...], preferred_element_type=jnp.float32)
    w = jnp.sum((alx * g).reshape(bsz, KNN_K, d2), axis=1)
    acc = w[:, 0:oc]
    for hh in range(1, heads):
        acc = acc + w[:, hh * oc:(hh + 1) * oc]
    z_ref[...] = acc * (1.0 / heads) + b2_ref[...]


def _gat_layer2(g2h, w2l, xr2, a2s, b2s, b2r, heads, bsz=128):
    npad, d2 = xr2.shape
    d = g2h.shape[1]
    oc = d2 // heads
    grid = npad // bsz
    return pl.pallas_call(
        functools.partial(_gat2_body, bsz, heads),
        grid=(grid,),
        in_specs=[
            pl.BlockSpec((bsz * KNN_K, d), lambda i: (i, 0)),
            pl.BlockSpec(w2l.shape, lambda i: (0, 0)),
            pl.BlockSpec((bsz, d2), lambda i: (i, 0)),
            pl.BlockSpec(a2s.shape, lambda i: (0, 0)),
            pl.BlockSpec(b2s.shape, lambda i: (0, 0)),
            pl.BlockSpec(b2r.shape, lambda i: (0, 0)),
        ],
        out_specs=pl.BlockSpec((bsz, oc), lambda i: (i, 0)),
        out_shape=jax.ShapeDtypeStruct((npad, oc), jnp.float32),
    )(g2h, w2l, xr2, a2s, b2s, b2r)


def _spread_att(a, nh_pad=8):
    heads, oc = a.shape
    eye = jnp.eye(heads, nh_pad, dtype=a.dtype)
    return (a[:, :, None] * eye[:, None, :]).reshape(heads * oc, nh_pad)


def _head_broadcast(heads, oc, nh_pad=8):
    return jnp.repeat(jnp.eye(nh_pad, heads, dtype=jnp.float32), oc, axis=1)


def kernel(x, W1l, W1r, a1, b1, W2l, W2r, a2, b2):
    n, d = x.shape
    heads, oc1 = a1.shape
    npad = ((n + 1023) // 1024) * 1024
    xp = jnp.pad(x, ((0, npad - n), (0, 0)))

    zn, xl1, xr1 = _normalize_and_project(xp, W1l, W1r)
    idx = _knn_topk(zn, zn.T, n)
    idx_flat = idx.reshape(-1)

    g1 = _sc_gather(xl1, idx_flat)
    a1s = _spread_att(a1)
    b1s = _head_broadcast(heads, oc1)
    h, xr2 = _gat_layer1(g1, xr1, a1s, b1s, b1.reshape(1, -1), W2r)

    g2h = _sc_gather(h, idx_flat)
    a2s = _spread_att(a2)
    b2s = _head_broadcast(heads, a2.shape[1])
    z = _gat_layer2(g2h, W2l, xr2, a2s, b2s, b2.reshape(1, -1), heads)
    return (x, z[:n])
